# baseline (speedup 1.0000x reference)
import jax
import jax.numpy as jnp
from jax.experimental import pallas as pl
from jax.experimental.pallas import tpu as pltpu

NETS = 100
D = 64
BATCH = 1024
NB = 20
PACK = 4
BT = 512

_CMP = jax.lax.Precision.DEFAULT
_APPLY = jax.lax.Precision.DEFAULT


def _mlp_kernel(x_ref, w0_ref, w1_ref, w2_ref, o_ref):
    ms = []
    for g in range(NB):
        m = jnp.dot(
            w2_ref[g],
            jnp.dot(w1_ref[g], w0_ref[g], precision=_CMP),
            precision=_CMP,
        )
        ms.append(m)
    for g in range(NB):
        o_ref[D * g : D * (g + 1), :] = jnp.dot(
            ms[g], x_ref[D * g : D * (g + 1), :], precision=_APPLY
        )


def kernel(x, w0, w1, w2):
    w0r = w0.reshape(NETS, D, D)
    w1r = w1.reshape(NETS, D, D)
    w2r = w2.reshape(NETS, D, D)
    grid = (NETS // NB, BATCH // BT)
    out = pl.pallas_call(
        _mlp_kernel,
        grid=grid,
        in_specs=[
            pl.BlockSpec((NB * D, BT), lambda i, j: (i, j)),
            pl.BlockSpec((NB, D, D), lambda i, j: (i, 0, 0)),
            pl.BlockSpec((NB, D, D), lambda i, j: (i, 0, 0)),
            pl.BlockSpec((NB, D, D), lambda i, j: (i, 0, 0)),
        ],
        out_specs=pl.BlockSpec((NB * D, BT), lambda i, j: (i, j)),
        out_shape=jax.ShapeDtypeStruct((NETS * D, BATCH), jnp.float32),
        compiler_params=pltpu.CompilerParams(
            dimension_semantics=("parallel", "parallel"),
        ),
    )(x, w0r, w1r, w2r)
    return out


# speedup vs baseline: 1.1322x; 1.1322x over previous
import jax
import jax.numpy as jnp
from jax.experimental import pallas as pl
from jax.experimental.pallas import tpu as pltpu

NETS = 100
D = 64
BATCH = 1024
NB = 25
PACK = 4
BT = 1024

_CMP = jax.lax.Precision.DEFAULT
_APPLY = jax.lax.Precision.DEFAULT


def _mlp_kernel(x_ref, w0_ref, w1_ref, w2_ref, o_ref):
    ms = []
    for g in range(NB):
        m = jnp.dot(
            w2_ref[g],
            jnp.dot(w1_ref[g], w0_ref[g], precision=_CMP),
            precision=_CMP,
        )
        ms.append(m)
    for g in range(NB):
        o_ref[D * g : D * (g + 1), :] = jnp.dot(
            ms[g], x_ref[D * g : D * (g + 1), :], precision=_APPLY
        )


def kernel(x, w0, w1, w2):
    w0r = w0.reshape(NETS, D, D)
    w1r = w1.reshape(NETS, D, D)
    w2r = w2.reshape(NETS, D, D)
    grid = (NETS // NB, BATCH // BT)
    out = pl.pallas_call(
        _mlp_kernel,
        grid=grid,
        in_specs=[
            pl.BlockSpec((NB * D, BT), lambda i, j: (i, j)),
            pl.BlockSpec((NB, D, D), lambda i, j: (i, 0, 0)),
            pl.BlockSpec((NB, D, D), lambda i, j: (i, 0, 0)),
            pl.BlockSpec((NB, D, D), lambda i, j: (i, 0, 0)),
        ],
        out_specs=pl.BlockSpec((NB * D, BT), lambda i, j: (i, j)),
        out_shape=jax.ShapeDtypeStruct((NETS * D, BATCH), jnp.float32),
        compiler_params=pltpu.CompilerParams(
            dimension_semantics=("parallel", "parallel"),
        ),
    )(x, w0r, w1r, w2r)
    return out
